# Initial kernel scaffold; baseline (speedup 1.0000x reference)
import jax
import jax.numpy as jnp
from jax.experimental import pallas as pl


def kernel(edge_feats, edge_ts, edge_types, time_freqs, W_all, b_all, type_emb):
    raise NotImplementedError("write your pallas kernel here")


# speedup vs baseline: 2.7294x; 2.7294x over previous
import functools

import jax
import jax.numpy as jnp
from jax.experimental import pallas as pl
from jax.experimental.pallas import tpu as pltpu

_NUM_TYPES = 4
_TIME = 100
_FEAT = 128
_OUT = 128
_ROWS = 2000


def _encode_block(ts_ref, tp_ref, feats_ref, freqs_ref, w_ref, bias_ref, out_ref):
    ts = ts_ref[...]
    tp = tp_ref[...]
    feats = feats_ref[...]

    frow = jnp.zeros((ts.shape[0], _TIME), dtype=jnp.float32)
    masks = []
    for i in range(_NUM_TYPES):
        m = (tp == i).astype(jnp.float32)
        masks.append(m)
        frow = frow + m * freqs_ref[i : i + 1, :]
    temb = jnp.cos(ts * frow)

    x = jnp.concatenate(
        [feats.astype(jnp.bfloat16), temb.astype(jnp.bfloat16)], axis=1
    )
    g = jnp.dot(x, w_ref[...], preferred_element_type=jnp.float32)

    acc = jnp.zeros((ts.shape[0], _OUT), dtype=jnp.float32)
    for i in range(_NUM_TYPES):
        acc = acc + masks[i] * (g[:, i * _OUT : (i + 1) * _OUT] + bias_ref[i : i + 1, :])
    out_ref[...] = acc


@functools.partial(jax.jit, static_argnames=())
def kernel(edge_feats, edge_ts, edge_types, time_freqs, W_all, b_all, type_emb):
    n = edge_feats.shape[0]
    nb = n // _ROWS
    ts2 = edge_ts.reshape(n, 1)
    tp2 = edge_types.reshape(n, 1).astype(jnp.int32)
    w_cat = jnp.transpose(W_all, (1, 0, 2)).reshape(_FEAT + _TIME, _NUM_TYPES * _OUT)
    w_cat = w_cat.astype(jnp.bfloat16)
    bias = (b_all + type_emb).astype(jnp.float32)

    return pl.pallas_call(
        _encode_block,
        grid=(nb,),
        in_specs=[
            pl.BlockSpec((_ROWS, 1), lambda i: (i, 0)),
            pl.BlockSpec((_ROWS, 1), lambda i: (i, 0)),
            pl.BlockSpec((_ROWS, _FEAT), lambda i: (i, 0)),
            pl.BlockSpec((_NUM_TYPES, _TIME), lambda i: (0, 0)),
            pl.BlockSpec((_FEAT + _TIME, _NUM_TYPES * _OUT), lambda i: (0, 0)),
            pl.BlockSpec((_NUM_TYPES, _OUT), lambda i: (0, 0)),
        ],
        out_specs=pl.BlockSpec((_ROWS, _OUT), lambda i: (i, 0)),
        out_shape=jax.ShapeDtypeStruct((n, _OUT), jnp.float32),
        compiler_params=pltpu.CompilerParams(
            dimension_semantics=("arbitrary",),
        ),
    )(ts2, tp2, edge_feats, time_freqs, w_cat, bias)
